# Initial kernel scaffold; baseline (speedup 1.0000x reference)
import jax
import jax.numpy as jnp
from jax.experimental import pallas as pl


def kernel(node_feat, W1, b1, W2, b2, dense):
    raise NotImplementedError("write your pallas kernel here")


# speedup vs baseline: 24.3432x; 24.3432x over previous
import jax
import jax.numpy as jnp
from jax.experimental import pallas as pl
from jax.experimental.pallas import tpu as pltpu

N = 10000
D_IN = 128
D_HID = 64
D_OUT = 32
TOP_K = 32
ROWS = 200


def _bit_i32(bit: int) -> jnp.int32:
    v = 1 << bit
    if v >= 2 ** 31:
        v -= 2 ** 32
    return jnp.int32(v)


def _mlp_body(x_ref, w1_ref, b1_ref, w2_ref, b2_ref, h_ref):
    x = x_ref[...]
    h1 = jax.lax.dot_general(x, w1_ref[...], (((1,), (0,)), ((), ())),
                             preferred_element_type=jnp.float32)
    h1 = jnp.maximum(h1 + b1_ref[...], 0.0)
    h2 = jax.lax.dot_general(h1, w2_ref[...], (((1,), (0,)), ((), ())),
                             preferred_element_type=jnp.float32)
    h_ref[...] = h2 + b2_ref[...]


def _topo_body(hb_ref, ha_ref, o_ref):
    hb = hb_ref[...]
    ha = ha_ref[...]
    sim = jax.lax.dot_general(hb, ha, (((1,), (1,)), ((), ())),
                              preferred_element_type=jnp.float32)

    ikey = jax.lax.bitcast_convert_type(sim, jnp.int32)
    skey = jnp.where(ikey >= 0, ikey, ikey ^ jnp.int32(0x7FFFFFFF))

    mini = jnp.int32(-2 ** 31)
    p = jnp.zeros((ROWS, 1), jnp.int32)
    for bit in range(31, -1, -1):
        c = p | _bit_i32(bit)
        cs = c ^ mini
        cnt = jnp.sum((skey >= cs).astype(jnp.int32), axis=1, keepdims=True)
        p = jnp.where(cnt >= TOP_K, c, p)
    thresh_key = p ^ mini

    mask = skey >= thresh_key
    rowmax = jnp.max(sim, axis=1, keepdims=True)
    e = jnp.where(mask, jnp.exp(sim - rowmax), 0.0)
    denom = jnp.sum(e, axis=1, keepdims=True)
    o_ref[...] = e / denom


def kernel(node_feat, W1, b1, W2, b2, dense):
    del dense
    b1r = b1.reshape(1, D_HID)
    b2r = b2.reshape(1, D_OUT)
    grid = N // ROWS

    h = pl.pallas_call(
        _mlp_body,
        grid=(grid,),
        in_specs=[
            pl.BlockSpec((ROWS, D_IN), lambda i: (i, 0)),
            pl.BlockSpec((D_IN, D_HID), lambda i: (0, 0)),
            pl.BlockSpec((1, D_HID), lambda i: (0, 0)),
            pl.BlockSpec((D_HID, D_OUT), lambda i: (0, 0)),
            pl.BlockSpec((1, D_OUT), lambda i: (0, 0)),
        ],
        out_specs=pl.BlockSpec((ROWS, D_OUT), lambda i: (i, 0)),
        out_shape=jax.ShapeDtypeStruct((N, D_OUT), jnp.float32),
        compiler_params=pltpu.CompilerParams(
            dimension_semantics=("parallel",)),
    )(node_feat, W1, b1r, W2, b2r)

    graph_topo = pl.pallas_call(
        _topo_body,
        grid=(grid,),
        in_specs=[
            pl.BlockSpec((ROWS, D_OUT), lambda i: (i, 0)),
            pl.BlockSpec((N, D_OUT), lambda i: (0, 0)),
        ],
        out_specs=pl.BlockSpec((ROWS, N), lambda i: (i, 0)),
        out_shape=jax.ShapeDtypeStruct((N, N), jnp.float32),
        compiler_params=pltpu.CompilerParams(
            dimension_semantics=("parallel",)),
    )(h, h)

    return graph_topo
